# baseline (speedup 1.0000x reference)
import functools

import jax
import jax.numpy as jnp
from jax import lax
from jax.experimental import pallas as pl
from jax.experimental.pallas import tpu as pltpu
from jax.experimental.pallas import tpu_sc as plsc

_K = 1024
_D = 64
_BLK = 1024


def _argmin_body(x_ref, w_ref, idx_ref):
    x = x_ref[...]
    w = w_ref[...]
    prod = lax.dot_general(
        x, w, (((1,), (1,)), ((), ())),
        preferred_element_type=jnp.float32) * 2.0
    xsq = jnp.sum(x * x, axis=1, keepdims=True)
    esq = jnp.sum(w * w, axis=1)
    d = (xsq + esq[None, :]) - prod
    m = jnp.min(d, axis=1, keepdims=True)
    ids = lax.broadcasted_iota(jnp.int32, d.shape, 1)
    idx_ref[...] = jnp.min(jnp.where(d == m, ids, _K), axis=1)


def _compute_indices(flat, W):
    n = flat.shape[0]
    return pl.pallas_call(
        _argmin_body,
        grid=(n // _BLK,),
        in_specs=[
            pl.BlockSpec((_BLK, _D), lambda i: (i, 0)),
            pl.BlockSpec((_K, _D), lambda i: (0, 0)),
        ],
        out_specs=pl.BlockSpec((_BLK,), lambda i: (i,)),
        out_shape=jax.ShapeDtypeStruct((n,), jnp.int32),
    )(flat, W)


def _gather_rows(W, idx):
    n = idx.shape[0]
    nw = 32
    bpw = n // nw
    mesh = plsc.VectorSubcoreMesh(core_axis_name="c", subcore_axis_name="s")

    @functools.partial(
        pl.kernel, mesh=mesh,
        out_type=jax.ShapeDtypeStruct((n, _D), jnp.float32),
        compiler_params=pltpu.CompilerParams(use_tc_tiling_on_sc=False),
        scratch_types=[
            pltpu.VMEM((bpw,), jnp.int32),
            pltpu.VMEM((bpw, _D), jnp.float32),
            pltpu.SemaphoreType.DMA,
        ],
    )
    def k(w_hbm, idx_hbm, out_hbm, idx_v, rows_v, sem):
        wid = lax.axis_index("s") * 2 + lax.axis_index("c")
        base = wid * bpw
        pltpu.sync_copy(idx_hbm.at[pl.ds(base, bpw)], idx_v)
        pltpu.async_copy(w_hbm.at[idx_v], rows_v, sem).wait()
        pltpu.sync_copy(rows_v, out_hbm.at[pl.ds(base, bpw)])

    return k(W, idx)


def kernel(x, W):
    n = x.shape[0] * x.shape[1]
    flat = x.reshape(n, _D)
    idx = _compute_indices(flat, W)
    quantized = _gather_rows(W, idx).reshape(x.shape)
    quantized_with_grad = x + lax.stop_gradient(quantized - x)
    return (quantized_with_grad, quantized, idx)


# speedup vs baseline: 1.1618x; 1.1618x over previous
import functools

import jax
import jax.numpy as jnp
from jax import lax
from jax.experimental import pallas as pl
from jax.experimental.pallas import tpu as pltpu
from jax.experimental.pallas import tpu_sc as plsc

_K = 1024
_D = 64
_BLK = 1024


def _argmin_body(x_ref, w_ref, idx_ref):
    x = x_ref[...]
    w = w_ref[...]
    prod = lax.dot_general(
        x, w, (((1,), (1,)), ((), ())),
        preferred_element_type=jnp.float32) * 2.0
    xsq = jnp.sum(x * x, axis=1, keepdims=True)
    esq = jnp.sum(w * w, axis=1)
    d = (xsq + esq[None, :]) - prod
    idx_ref[...] = jnp.argmin(d, axis=1).astype(jnp.int32)


def _compute_indices(flat, W):
    n = flat.shape[0]
    return pl.pallas_call(
        _argmin_body,
        grid=(n // _BLK,),
        in_specs=[
            pl.BlockSpec((_BLK, _D), lambda i: (i, 0)),
            pl.BlockSpec((_K, _D), lambda i: (0, 0)),
        ],
        out_specs=pl.BlockSpec((_BLK,), lambda i: (i,)),
        out_shape=jax.ShapeDtypeStruct((n,), jnp.int32),
    )(flat, W)


def _gather_rows(W, idx):
    n = idx.shape[0]
    nw = 32
    bpw = n // nw
    mesh = plsc.VectorSubcoreMesh(core_axis_name="c", subcore_axis_name="s")

    @functools.partial(
        pl.kernel, mesh=mesh,
        out_type=jax.ShapeDtypeStruct((n, _D), jnp.float32),
        compiler_params=pltpu.CompilerParams(use_tc_tiling_on_sc=False),
        scratch_types=[
            pltpu.VMEM((bpw,), jnp.int32),
            pltpu.VMEM((bpw, _D), jnp.float32),
            pltpu.SemaphoreType.DMA,
        ],
    )
    def k(w_hbm, idx_hbm, out_hbm, idx_v, rows_v, sem):
        wid = lax.axis_index("s") * 2 + lax.axis_index("c")
        base = wid * bpw
        pltpu.sync_copy(idx_hbm.at[pl.ds(base, bpw)], idx_v)
        pltpu.async_copy(w_hbm.at[idx_v], rows_v, sem).wait()
        pltpu.sync_copy(rows_v, out_hbm.at[pl.ds(base, bpw)])

    return k(W, idx)


def kernel(x, W):
    n = x.shape[0] * x.shape[1]
    flat = x.reshape(n, _D)
    idx = _compute_indices(flat, W)
    quantized = _gather_rows(W, idx).reshape(x.shape)
    quantized_with_grad = x + lax.stop_gradient(quantized - x)
    return (quantized_with_grad, quantized, idx)
